# baseline (speedup 1.0000x reference)
import functools

import jax
import jax.numpy as jnp
from jax import lax
from jax.experimental import pallas as pl
from jax.experimental.pallas import tpu as pltpu
from jax.experimental.pallas import tpu_sc as plsc


def _gather_rows(xf, perm_i32, B, S, D):
    info = plsc.get_sparse_core_info()
    NC, NS, L = info.num_cores, info.num_subcores, info.num_lanes
    rows_total = B * S
    rows_per_sc = rows_total // NC
    CH = 16
    RND = NS * CH
    n_r = rows_per_sc // RND
    NB = 3
    NG = 4
    P = 3

    mesh = plsc.VectorSubcoreMesh(core_axis_name="c", subcore_axis_name="s")

    @functools.partial(
        pl.kernel,
        mesh=mesh,
        out_type=jax.ShapeDtypeStruct((rows_total, D), jnp.float32),
        scratch_types=[
            pltpu.VMEM((n_r * CH,), jnp.int32),
        ] + [pltpu.VMEM((CH, D), jnp.float32) for _ in range(NG)]
          + [pltpu.VMEM_SHARED((RND, D), jnp.float32) for _ in range(NB)] + [
            pltpu.SemaphoreType.DMA,
            pltpu.SemaphoreType.DMA,
            pltpu.SemaphoreType.DMA,
        ],
    )
    def _k(x_hbm, perm_hbm, out_hbm, idx_v, *rest):
        gbufs = rest[:NG]
        spbufs = rest[NG:NG + NB]
        sem_g, sem_x, sem_s = rest[NG + NB:]
        cid = lax.axis_index("c")
        sid = lax.axis_index("s")
        sc_base = cid * rows_per_sc

        rounds_per_b = S // RND
        for r in range(n_r):
            seq0 = pl.multiple_of((r % rounds_per_b) * RND + sid * CH, CH)
            pltpu.sync_copy(perm_hbm.at[pl.ds(seq0, CH)],
                            idx_v.at[pl.ds(r * CH, CH)])
        for r in range(n_r):
            boff = ((sc_base + r * RND) // S) * S
            for j in range(CH // L):
                sl = pl.ds(r * CH + j * L, L)
                idx_v[sl] = idx_v[sl] + boff

        def gather_start(r):
            return pltpu.async_copy(
                x_hbm.at[idx_v.at[pl.ds(r * CH, CH)]], gbufs[r % NG], sem_g
            )

        g = [None] * n_r
        x = [None] * n_r
        for r in range(min(P, n_r)):
            g[r] = gather_start(r)
        for r in range(n_r):
            g[r].wait()
            x[r] = pltpu.async_copy(
                gbufs[r % NG], spbufs[r % NB].at[pl.ds(pl.multiple_of(sid * CH, CH), CH)], sem_x
            )
            if r + P < n_r:
                g[r + P] = gather_start(r + P)
            x[r].wait()
            plsc.subcore_barrier()

            @pl.when(sid == 0)
            def _scatter(r=r):
                pltpu.async_copy(
                    spbufs[r % NB],
                    out_hbm.at[pl.ds(pl.multiple_of(sc_base + r * RND, RND), RND)],
                    sem_s,
                )
                if r - (NB - 1) >= 0:
                    pltpu.make_async_copy(
                        spbufs[0], out_hbm.at[pl.ds(0, RND)], sem_s
                    ).wait()

            plsc.subcore_barrier()

        @pl.when(sid == 0)
        def _drain():
            for _ in range(min(NB - 1, n_r)):
                pltpu.make_async_copy(
                    spbufs[0], out_hbm.at[pl.ds(0, RND)], sem_s
                ).wait()

    return _k(xf, perm_i32)


def kernel(x, perm):
    B, S, D = x.shape
    xf = x.reshape(B * S, D)
    out = _gather_rows(xf, perm.astype(jnp.int32), B, S, D)
    return out.reshape(B, S, D)
